# baseline (speedup 1.0000x reference)
import jax
import jax.numpy as jnp
from jax import lax
from jax.experimental import pallas as pl
from jax.experimental.pallas import tpu as pltpu
from jax.experimental.pallas import tpu_sc as plsc

NC = 2
NS = 16
L = 16
NW = NC * NS

B = 16384
D = 32
V = 1000000
ROWS_PER_PACK = 128 // D
Q = V // ROWS_PER_PACK
BPW = B // NW
PB = 256
NPASS = BPW // PB
CHUNK = 128
GROUPS = PB // L

RC = 1024
RQ = RC // ROWS_PER_PACK
NFULL = V // RC
TAILA_C0 = NFULL * RC
TAILA_NC = 512
TAILB_W0 = V - 64
TAILB_Q0 = (V - 64) // ROWS_PER_PACK


def _repack_body(ut, it, tu, ti, up_hbm, ip_hbm, slab, orow, sem):
    wid = lax.axis_index("s") * NC + lax.axis_index("c")
    lane = lax.iota(jnp.int32, L)

    lane43 = lax.shift_left(lane, 3) + jnp.bitwise_and(lane, 3)
    lane2 = lax.shift_left(lane, 1)
    d_base = lax.shift_left(lax.shift_right_logical(lane, 2), 3)
    c_base = lax.shift_left(jnp.bitwise_and(lane, 3), 5) + d_base
    dims_dr = [d_base + dr for dr in range(8)]
    cols_dr = [c_base + dr for dr in range(8)]

    def extract(ncols, col_lo):
        @plsc.parallel_loop(0, (ncols // 128) * 32, unroll=8)
        def one_s(si):
            blk = lax.shift_right_logical(si, 5)
            s = jnp.bitwise_and(si, 31)
            c0 = col_lo + blk * 128
            q0 = lax.shift_right_logical(c0, 2)
            w = jnp.bitwise_and(lane43 + 4 * s, 127)
            srcc = c0 + w
            dstr = q0 + lax.shift_right_logical(w, 2)
            for dr in range(8):
                vals = plsc.load_gather(slab, [dims_dr[dr], srcc])
                plsc.store_scatter(orow, [dstr, cols_dr[dr]], vals)

    def do_table(tbl, out_hbm, tail):
        def chunk_iter(it_, carry):
            cid = it_ * NW + wid

            @pl.when(cid < NFULL)
            def _():
                col0 = cid * RC
                pltpu.sync_copy(tbl.at[:, pl.ds(col0, RC)],
                                slab.at[:, pl.ds(0, RC)])
                extract(RC, 0)
                pltpu.sync_copy(orow.at[pl.ds(0, RQ), :],
                                out_hbm.at[pl.ds(cid * RQ, RQ), :])
            return carry

        lax.fori_loop(0, NFULL // NW + 1, chunk_iter, 0)

        @pl.when(wid == 0)
        def _():
            pltpu.sync_copy(tbl.at[:, pl.ds(TAILA_C0, TAILA_NC)],
                            slab.at[:, pl.ds(0, TAILA_NC)])
            extract(TAILA_NC, 0)
            pltpu.sync_copy(orow.at[pl.ds(0, TAILA_NC // 4), :],
                            out_hbm.at[pl.ds(TAILA_C0 // 4, TAILA_NC // 4), :])

        @pl.when(wid == 1)
        def _():
            pltpu.sync_copy(tail, orow.at[pl.ds(0, 16), :])
            pltpu.sync_copy(orow.at[pl.ds(0, 16), :],
                            out_hbm.at[pl.ds(TAILB_Q0, 16), :])

    do_table(ut, up_hbm, tu)
    do_table(it, ip_hbm, ti)


def _bpr_body(ur, ir_, u_hbm, i_hbm, j_hbm, oi_hbm, oj_hbm,
              u_idx, i_idx, j_idx, u_q, i_q, j_q,
              u_rows, i_rows, j_rows, oi_v, oj_v, sem):
    wid = lax.axis_index("s") * NC + lax.axis_index("c")
    lane = lax.iota(jnp.int32, L)

    for p in range(NPASS):
        base = wid * BPW + p * PB
        pltpu.sync_copy(u_hbm.at[pl.ds(base, PB)], u_idx)
        pltpu.sync_copy(i_hbm.at[pl.ds(base, PB)], i_idx)
        pltpu.sync_copy(j_hbm.at[pl.ds(base, PB)], j_idx)

        for g in range(GROUPS):
            sl = pl.ds(g * L, L)
            u_q[sl] = lax.shift_right_logical(u_idx[sl], 2)
            i_q[sl] = lax.shift_right_logical(i_idx[sl], 2)
            j_q[sl] = lax.shift_right_logical(j_idx[sl], 2)

        copies = []
        for c in range(PB // CHUNK):
            sl = pl.ds(c * CHUNK, CHUNK)
            copies.append(pltpu.async_copy(ur.at[u_q.at[sl]], u_rows.at[sl, :], sem))
            copies.append(pltpu.async_copy(ir_.at[i_q.at[sl]], i_rows.at[sl, :], sem))
            copies.append(pltpu.async_copy(ir_.at[j_q.at[sl]], j_rows.at[sl, :], sem))
        for cp in copies:
            cp.wait()

        def group(g, carry):
            sl = pl.ds(g * L, L)
            su = lax.shift_left(jnp.bitwise_and(u_idx[sl], 3), 5)
            si = lax.shift_left(jnp.bitwise_and(i_idx[sl], 3), 5)
            sj = lax.shift_left(jnp.bitwise_and(j_idx[sl], 3), 5)
            acc_i = jnp.zeros((L,), jnp.float32)
            acc_j = jnp.zeros((L,), jnp.float32)
            for l in range(L):
                k = g * L + l
                ou = su[l]
                oi = si[l]
                oj = sj[l]
                u0 = u_rows[k, pl.ds(ou, L)]
                u1 = u_rows[k, pl.ds(ou + L, L)]
                i0 = i_rows[k, pl.ds(oi, L)]
                i1 = i_rows[k, pl.ds(oi + L, L)]
                j0 = j_rows[k, pl.ds(oj, L)]
                j1 = j_rows[k, pl.ds(oj + L, L)]
                pi_s = jnp.sum(u0 * i0 + u1 * i1)
                pj_s = jnp.sum(u0 * j0 + u1 * j1)
                m = lane == l
                acc_i = jnp.where(m, pi_s, acc_i)
                acc_j = jnp.where(m, pj_s, acc_j)
            osl = pl.ds(p * PB + g * L, L)
            oi_v[osl] = acc_i
            oj_v[osl] = acc_j
            return carry

        lax.fori_loop(0, GROUPS, group, 0)

    base = wid * BPW
    pltpu.sync_copy(oi_v, oi_hbm.at[pl.ds(base, BPW)])
    pltpu.sync_copy(oj_v, oj_hbm.at[pl.ds(base, BPW)])


def kernel(user_embd, item_embd, u, i, j):
    mesh = plsc.VectorSubcoreMesh(core_axis_name="c", subcore_axis_name="s")

    repack = pl.kernel(
        _repack_body,
        out_type=[
            jax.ShapeDtypeStruct((Q, 128), jnp.float32),
            jax.ShapeDtypeStruct((Q, 128), jnp.float32),
        ],
        mesh=mesh,
        scratch_types=[
            pltpu.VMEM((D, RC), jnp.float32),
            pltpu.VMEM((RQ, 128), jnp.float32),
            pltpu.SemaphoreType.DMA,
        ],
        compiler_params=pltpu.CompilerParams(
            needs_layout_passes=False, use_tc_tiling_on_sc=True),
    )

    gather_dot = pl.kernel(
        _bpr_body,
        out_type=[
            jax.ShapeDtypeStruct((B,), jnp.float32),
            jax.ShapeDtypeStruct((B,), jnp.float32),
        ],
        mesh=mesh,
        scratch_types=[
            pltpu.VMEM((PB,), jnp.int32),
            pltpu.VMEM((PB,), jnp.int32),
            pltpu.VMEM((PB,), jnp.int32),
            pltpu.VMEM((PB,), jnp.int32),
            pltpu.VMEM((PB,), jnp.int32),
            pltpu.VMEM((PB,), jnp.int32),
            pltpu.VMEM((PB, 128), jnp.float32),
            pltpu.VMEM((PB, 128), jnp.float32),
            pltpu.VMEM((PB, 128), jnp.float32),
            pltpu.VMEM((BPW,), jnp.float32),
            pltpu.VMEM((BPW,), jnp.float32),
            pltpu.SemaphoreType.DMA,
        ],
        compiler_params=pltpu.CompilerParams(
            needs_layout_passes=False, use_tc_tiling_on_sc=True),
    )

    ut = jnp.swapaxes(user_embd, 0, 1)
    it = jnp.swapaxes(item_embd, 0, 1)
    tu = user_embd[TAILB_Q0 * ROWS_PER_PACK:].reshape(16, 128)
    ti = item_embd[TAILB_Q0 * ROWS_PER_PACK:].reshape(16, 128)
    ur, ir_ = repack(ut, it, tu, ti)
    pi, pj = gather_dot(ur, ir_, u.astype(jnp.int32), i.astype(jnp.int32),
                        j.astype(jnp.int32))
    return pi, pj


# speedup vs baseline: 1.7976x; 1.7976x over previous
import jax
import jax.numpy as jnp
from jax import lax
from jax.experimental import pallas as pl
from jax.experimental.pallas import tpu as pltpu
from jax.experimental.pallas import tpu_sc as plsc

NC = 2
NS = 16
L = 16
NW = NC * NS

B = 16384
D = 32
V = 1000000
ROWS_PER_PACK = 128 // D
Q = V // ROWS_PER_PACK
BPW = B // NW
PB = 256
NPASS = BPW // PB
CHUNK = 128
GROUPS = PB // L

RC = 512
RQ = RC // ROWS_PER_PACK
NCH = 61
NFULL = NCH * NW
TAILA_C0 = NFULL * RC
TAILA_NC = 512
TAILB_W0 = V - 64
TAILB_Q0 = (V - 64) // ROWS_PER_PACK


def _repack_body(ut, it, tu, ti, up_hbm, ip_hbm, slab0, slab1, orow0, orow1,
                 isem0, isem1, osem0, osem1):
    wid = lax.axis_index("s") * NC + lax.axis_index("c")
    lane = lax.iota(jnp.int32, L)

    lane43 = lax.shift_left(lane, 3) + jnp.bitwise_and(lane, 3)
    lane2 = lax.shift_left(lane, 1)
    d_base = lax.shift_left(lax.shift_right_logical(lane, 2), 3)
    c_base = lax.shift_left(jnp.bitwise_and(lane, 3), 5) + d_base
    dims_dr = [d_base + dr for dr in range(8)]
    cols_dr = [c_base + dr for dr in range(8)]

    def extract(slab, orow, ncols, col_lo):
        def one_blk(blk, carry):
            c0 = col_lo + blk * 128
            q0 = lax.shift_right_logical(c0, 2)

            @plsc.parallel_loop(0, 32, unroll=4)
            def one_s(s):
                w = jnp.bitwise_and(lane43 + 4 * s, 127)
                srcc = c0 + w
                dstr = q0 + lax.shift_right_logical(w, 2)
                for dr in range(8):
                    vals = plsc.load_gather(slab, [dims_dr[dr], srcc])
                    plsc.store_scatter(orow, [dstr, cols_dr[dr]], vals)
            return carry
        lax.fori_loop(0, ncols // 128, one_blk, 0)

    def do_table(tbl, out_hbm, tail):
        def cslice(chunk_it):
            return tbl.at[:, pl.ds((chunk_it * NW + wid) * RC, RC)]

        def oslice(chunk_it):
            return out_hbm.at[pl.ds((chunk_it * NW + wid) * RQ, RQ), :]

        def wait_in(slab, isem):
            pltpu.make_async_copy(tbl.at[:, pl.ds(0, RC)], slab, isem).wait()

        def wait_out(orow, osem):
            pltpu.make_async_copy(orow, out_hbm.at[pl.ds(0, RQ), :],
                                  osem).wait()

        pltpu.async_copy(cslice(0), slab0, isem0)

        def step(p, carry):
            wait_in(slab0, isem0)
            pltpu.async_copy(cslice(2 * p + 1), slab1, isem1)

            @pl.when(p >= 1)
            def _():
                wait_out(orow0, osem0)
            extract(slab0, orow0, RC, 0)
            pltpu.async_copy(orow0, oslice(2 * p), osem0)

            wait_in(slab1, isem1)
            pltpu.async_copy(cslice(2 * p + 2), slab0, isem0)

            @pl.when(p >= 1)
            def _():
                wait_out(orow1, osem1)
            extract(slab1, orow1, RC, 0)
            pltpu.async_copy(orow1, oslice(2 * p + 1), osem1)
            return carry

        lax.fori_loop(0, (NCH - 1) // 2, step, 0)

        wait_in(slab0, isem0)
        wait_out(orow0, osem0)
        extract(slab0, orow0, RC, 0)
        pltpu.async_copy(orow0, oslice(NCH - 1), osem0)
        wait_out(orow0, osem0)
        wait_out(orow1, osem1)

        @pl.when(wid == 0)
        def _():
            pltpu.sync_copy(tbl.at[:, pl.ds(TAILA_C0, TAILA_NC)],
                            slab0.at[:, pl.ds(0, TAILA_NC)])
            extract(slab0, orow0, TAILA_NC, 0)
            pltpu.sync_copy(orow0.at[pl.ds(0, TAILA_NC // 4), :],
                            out_hbm.at[pl.ds(TAILA_C0 // 4, TAILA_NC // 4), :])

        @pl.when(wid == 1)
        def _():
            pltpu.sync_copy(tail, orow0.at[pl.ds(0, 16), :])
            pltpu.sync_copy(orow0.at[pl.ds(0, 16), :],
                            out_hbm.at[pl.ds(TAILB_Q0, 16), :])

    do_table(ut, up_hbm, tu)
    do_table(it, ip_hbm, ti)


def _bpr_body(ur, ir_, u_hbm, i_hbm, j_hbm, oi_hbm, oj_hbm,
              u_idx, i_idx, j_idx, u_q, i_q, j_q,
              u_rows, i_rows, j_rows, oi_v, oj_v, sem):
    wid = lax.axis_index("s") * NC + lax.axis_index("c")
    lane = lax.iota(jnp.int32, L)

    for p in range(NPASS):
        base = wid * BPW + p * PB
        pltpu.sync_copy(u_hbm.at[pl.ds(base, PB)], u_idx)
        pltpu.sync_copy(i_hbm.at[pl.ds(base, PB)], i_idx)
        pltpu.sync_copy(j_hbm.at[pl.ds(base, PB)], j_idx)

        for g in range(GROUPS):
            sl = pl.ds(g * L, L)
            u_q[sl] = lax.shift_right_logical(u_idx[sl], 2)
            i_q[sl] = lax.shift_right_logical(i_idx[sl], 2)
            j_q[sl] = lax.shift_right_logical(j_idx[sl], 2)

        copies = []
        for c in range(PB // CHUNK):
            sl = pl.ds(c * CHUNK, CHUNK)
            copies.append(pltpu.async_copy(ur.at[u_q.at[sl]], u_rows.at[sl, :], sem))
            copies.append(pltpu.async_copy(ir_.at[i_q.at[sl]], i_rows.at[sl, :], sem))
            copies.append(pltpu.async_copy(ir_.at[j_q.at[sl]], j_rows.at[sl, :], sem))
        for cp in copies:
            cp.wait()

        def group(g, carry):
            sl = pl.ds(g * L, L)
            su = lax.shift_left(jnp.bitwise_and(u_idx[sl], 3), 5)
            si = lax.shift_left(jnp.bitwise_and(i_idx[sl], 3), 5)
            sj = lax.shift_left(jnp.bitwise_and(j_idx[sl], 3), 5)
            acc_i = jnp.zeros((L,), jnp.float32)
            acc_j = jnp.zeros((L,), jnp.float32)
            for l in range(L):
                k = g * L + l
                ou = su[l]
                oi = si[l]
                oj = sj[l]
                u0 = u_rows[k, pl.ds(ou, L)]
                u1 = u_rows[k, pl.ds(ou + L, L)]
                i0 = i_rows[k, pl.ds(oi, L)]
                i1 = i_rows[k, pl.ds(oi + L, L)]
                j0 = j_rows[k, pl.ds(oj, L)]
                j1 = j_rows[k, pl.ds(oj + L, L)]
                pi_s = jnp.sum(u0 * i0 + u1 * i1)
                pj_s = jnp.sum(u0 * j0 + u1 * j1)
                m = lane == l
                acc_i = jnp.where(m, pi_s, acc_i)
                acc_j = jnp.where(m, pj_s, acc_j)
            osl = pl.ds(p * PB + g * L, L)
            oi_v[osl] = acc_i
            oj_v[osl] = acc_j
            return carry

        lax.fori_loop(0, GROUPS, group, 0)

    base = wid * BPW
    pltpu.sync_copy(oi_v, oi_hbm.at[pl.ds(base, BPW)])
    pltpu.sync_copy(oj_v, oj_hbm.at[pl.ds(base, BPW)])


def kernel(user_embd, item_embd, u, i, j):
    mesh = plsc.VectorSubcoreMesh(core_axis_name="c", subcore_axis_name="s")

    repack = pl.kernel(
        _repack_body,
        out_type=[
            jax.ShapeDtypeStruct((Q, 128), jnp.float32),
            jax.ShapeDtypeStruct((Q, 128), jnp.float32),
        ],
        mesh=mesh,
        scratch_types=[
            pltpu.VMEM((D, RC), jnp.float32),
            pltpu.VMEM((D, RC), jnp.float32),
            pltpu.VMEM((RQ, 128), jnp.float32),
            pltpu.VMEM((RQ, 128), jnp.float32),
            pltpu.SemaphoreType.DMA,
            pltpu.SemaphoreType.DMA,
            pltpu.SemaphoreType.DMA,
            pltpu.SemaphoreType.DMA,
        ],
        compiler_params=pltpu.CompilerParams(
            needs_layout_passes=False, use_tc_tiling_on_sc=True),
    )

    gather_dot = pl.kernel(
        _bpr_body,
        out_type=[
            jax.ShapeDtypeStruct((B,), jnp.float32),
            jax.ShapeDtypeStruct((B,), jnp.float32),
        ],
        mesh=mesh,
        scratch_types=[
            pltpu.VMEM((PB,), jnp.int32),
            pltpu.VMEM((PB,), jnp.int32),
            pltpu.VMEM((PB,), jnp.int32),
            pltpu.VMEM((PB,), jnp.int32),
            pltpu.VMEM((PB,), jnp.int32),
            pltpu.VMEM((PB,), jnp.int32),
            pltpu.VMEM((PB, 128), jnp.float32),
            pltpu.VMEM((PB, 128), jnp.float32),
            pltpu.VMEM((PB, 128), jnp.float32),
            pltpu.VMEM((BPW,), jnp.float32),
            pltpu.VMEM((BPW,), jnp.float32),
            pltpu.SemaphoreType.DMA,
        ],
        compiler_params=pltpu.CompilerParams(
            needs_layout_passes=False, use_tc_tiling_on_sc=True),
    )

    ut = jnp.swapaxes(user_embd, 0, 1)
    it = jnp.swapaxes(item_embd, 0, 1)
    tu = user_embd[TAILB_Q0 * ROWS_PER_PACK:].reshape(16, 128)
    ti = item_embd[TAILB_Q0 * ROWS_PER_PACK:].reshape(16, 128)
    ur, ir_ = repack(ut, it, tu, ti)
    pi, pj = gather_dot(ur, ir_, u.astype(jnp.int32), i.astype(jnp.int32),
                        j.astype(jnp.int32))
    return pi, pj
